# baseline (speedup 1.0000x reference)
import jax
import jax.numpy as jnp
from jax.experimental import pallas as pl
from jax.experimental.pallas import tpu as pltpu

P = 256
D = 16
ED = 16
BB = 2


def _dot(a, b, dims=((1,), (0,))):
    return jax.lax.dot_general(a, b, (dims, ((), ())),
                               preferred_element_type=jnp.float32)


def _gin_kernel(nodes_ref, nodesT_ref, We1_ref, We1T_ref, We2T_ref, wd_ref,
                be_ref, bec_ref, Wn1T_ref, Wn2T_ref, bnc_ref, out_ref, agg_scr):
    rows = jax.lax.broadcasted_iota(jnp.int32, (P, P), 0)
    cols = jax.lax.broadcasted_iota(jnp.int32, (P, P), 1)
    offdiag = (rows != cols).astype(jnp.float32)
    ones_col = jnp.ones((P, 1), jnp.bfloat16)
    ones_row = jnp.ones((1, P), jnp.bfloat16)

    for i in range(BB):
        nodes = nodes_ref[i]
        nT = nodesT_ref[i]

        g = _dot(nodes, nT)
        sq_row = jnp.sum(nT * nT, axis=0, keepdims=True)
        sq_col = jnp.sum(nodes * nodes, axis=1, keepdims=True)
        dist = jnp.sqrt(jnp.maximum(sq_col + sq_row - 2.0 * g, 0.0)) * offdiag

        a2bf = (_dot(nodes, We1_ref[...]) + be_ref[...]).astype(jnp.bfloat16)
        cTbf = _dot(We2T_ref[...], nT).astype(jnp.bfloat16)

        for k in range(ED):
            lhs2 = jnp.concatenate([a2bf[:, k:k + 1], ones_col], axis=1)
            rhs2 = jnp.concatenate([ones_row, cTbf[k:k + 1, :]], axis=0)
            bc = _dot(lhs2, rhs2)
            m = jnp.maximum(dist * wd_ref[0, k] + bc, 0.0)
            mbf = m.astype(jnp.bfloat16)
            agg_scr[k:k + 1, :] = _dot(ones_row, mbf)

        diag = jnp.maximum(a2bf.T.astype(jnp.float32)
                           + cTbf.astype(jnp.float32), 0.0)
        aggT = agg_scr[...] - diag.astype(jnp.bfloat16).astype(jnp.float32)

        newT = (_dot(Wn1T_ref[...], aggT)
                + _dot(Wn2T_ref[...], nT)
                + bnc_ref[...])
        out_ref[i] = newT


def kernel(t, h, W_e, b_e, W_n, b_n):
    del t
    B = h.shape[0]
    nodes = h.reshape(B, P, D)
    nodesT = nodes.transpose(0, 2, 1)

    We1 = W_e[:D]
    We1T = We1.T
    We2T = W_e[D:2 * D].T
    wd = W_e[2 * D:2 * D + 1]
    be = b_e.reshape(1, ED)
    bec = b_e.reshape(ED, 1)
    Wn1T = W_n[:ED].T
    Wn2T = W_n[ED:].T
    bnc = b_n.reshape(D, 1)

    full = lambda shape: pl.BlockSpec(shape, lambda b: (0,) * len(shape))
    outT = pl.pallas_call(
        _gin_kernel,
        grid=(B // BB,),
        in_specs=[
            pl.BlockSpec((BB, P, D), lambda b: (b, 0, 0)),
            pl.BlockSpec((BB, D, P), lambda b: (b, 0, 0)),
            full((D, ED)), full((ED, D)), full((ED, D)), full((1, ED)),
            full((1, ED)), full((ED, 1)), full((D, ED)), full((D, D)),
            full((D, 1)),
        ],
        out_specs=pl.BlockSpec((BB, D, P), lambda b: (b, 0, 0)),
        out_shape=jax.ShapeDtypeStruct((B, D, P), jnp.float32),
        scratch_shapes=[pltpu.VMEM((ED, P), jnp.float32)],
        compiler_params=pltpu.CompilerParams(
            dimension_semantics=("parallel",)),
    )(nodes, nodesT, We1, We1T, We2T, wd, be, bec, Wn1T, Wn2T, bnc)

    return outT.transpose(0, 2, 1).reshape(B, P * D)


# speedup vs baseline: 3.2348x; 3.2348x over previous
import jax
import jax.numpy as jnp
from jax.experimental import pallas as pl
from jax.experimental.pallas import tpu as pltpu

P = 256
D = 16
ED = 16
BB = 2


def _dot(a, b, dims=((1,), (0,))):
    return jax.lax.dot_general(a, b, (dims, ((), ())),
                               preferred_element_type=jnp.float32)


def _gin_kernel(nodes_ref, nodesT_ref, We1_ref, We1T_ref, We2T_ref, wdbf_ref,
                be_ref, bec_ref, Wn1T_ref, Wn2T_ref, bnc_ref, out_ref, agg_scr):
    for i in range(BB):
        nodes = nodes_ref[i]
        nT = nodesT_ref[i]

        g = _dot(nodes, nT)
        sq_row = jnp.sum(nT * nT, axis=0, keepdims=True)
        sq_col = jnp.sum(nodes * nodes, axis=1, keepdims=True)
        dist = jnp.sqrt(jnp.maximum(sq_col + sq_row - 2.0 * g, 0.0))
        distbf = dist.astype(jnp.bfloat16)

        a2bf = (_dot(nodes, We1_ref[...]) + be_ref[...]).astype(jnp.bfloat16)
        cTbf = _dot(We2T_ref[...], nT).astype(jnp.bfloat16)

        for k in range(ED):
            wk = jax.lax.convert_element_type(wdbf_ref[0, k], jnp.bfloat16)
            m = distbf * wk + a2bf[:, k:k + 1] + cTbf[k:k + 1, :]
            m = jnp.maximum(m, jnp.bfloat16(0))
            s1 = m[:P // 2] + m[P // 2:]
            s2 = s1[:P // 4] + s1[P // 4:]
            s3 = s2[:P // 8] + s2[P // 8:]
            agg_scr[k:k + 1, :] = jnp.sum(s3.astype(jnp.float32),
                                          axis=0, keepdims=True)
        diag = jnp.maximum(jnp.transpose(a2bf) + cTbf, jnp.bfloat16(0))
        aggT = agg_scr[...] - diag.astype(jnp.float32)

        newT = (_dot(Wn1T_ref[...], aggT)
                + _dot(Wn2T_ref[...], nT)
                + bnc_ref[...])
        out_ref[i] = newT


def kernel(t, h, W_e, b_e, W_n, b_n):
    del t
    B = h.shape[0]
    nodes = h.reshape(B, P, D)
    nodesT = nodes.transpose(0, 2, 1)

    We1 = W_e[:D]
    We1T = We1.T
    We2T = W_e[D:2 * D].T
    wdbf = W_e[2 * D:2 * D + 1]
    be = b_e.reshape(1, ED)
    bec = b_e.reshape(ED, 1)
    Wn1T = W_n[:ED].T
    Wn2T = W_n[ED:].T
    bnc = b_n.reshape(D, 1)

    full = lambda shape: pl.BlockSpec(shape, lambda b: (0,) * len(shape))
    outT = pl.pallas_call(
        _gin_kernel,
        grid=(B // BB,),
        in_specs=[
            pl.BlockSpec((BB, P, D), lambda b: (b, 0, 0)),
            pl.BlockSpec((BB, D, P), lambda b: (b, 0, 0)),
            full((D, ED)), full((ED, D)), full((ED, D)), full((1, ED)),
            full((1, ED)), full((ED, 1)), full((D, ED)), full((D, D)),
            full((D, 1)),
        ],
        out_specs=pl.BlockSpec((BB, D, P), lambda b: (b, 0, 0)),
        out_shape=jax.ShapeDtypeStruct((B, D, P), jnp.float32),
        scratch_shapes=[pltpu.VMEM((ED, P), jnp.float32)],
        compiler_params=pltpu.CompilerParams(
            dimension_semantics=("parallel",)),
    )(nodes, nodesT, We1, We1T, We2T, wdbf, be, bec, Wn1T, Wn2T, bnc)

    return outT.transpose(0, 2, 1).reshape(B, P * D)


# speedup vs baseline: 3.2360x; 1.0004x over previous
import jax
import jax.numpy as jnp
from jax.experimental import pallas as pl
from jax.experimental.pallas import tpu as pltpu

P = 256
D = 16
ED = 16
BB = 2


def _dot(a, b, dims=((1,), (0,))):
    return jax.lax.dot_general(a, b, (dims, ((), ())),
                               preferred_element_type=jnp.float32)


def _gin_kernel(nodes_ref, nodesT_ref, We1_ref, We1T_ref, We2T_ref, wdbf_ref,
                be_ref, bec_ref, Wn1T_ref, Wn2T_ref, bnc_ref, out_ref, agg_scr):
    for i in range(BB):
        nodes = nodes_ref[i]
        nT = nodesT_ref[i]

        g = _dot(nodes, nT)
        sq_row = jnp.sum(nT * nT, axis=0, keepdims=True)
        sq_col = jnp.sum(nodes * nodes, axis=1, keepdims=True)
        dist = jnp.sqrt(jnp.abs(sq_col + sq_row - 2.0 * g))
        distbf = dist.astype(jnp.bfloat16)

        a2bf = (_dot(nodes, We1_ref[...]) + be_ref[...]).astype(jnp.bfloat16)
        cTbf = _dot(We2T_ref[...], nT).astype(jnp.bfloat16)

        for k in range(ED):
            wk = jax.lax.convert_element_type(wdbf_ref[0, k], jnp.bfloat16)
            m = distbf * wk + a2bf[:, k:k + 1] + cTbf[k:k + 1, :]
            m = jnp.maximum(m, jnp.bfloat16(0))
            s1 = m[:P // 2] + m[P // 2:]
            s2 = s1[:P // 4] + s1[P // 4:]
            s3 = s2[:P // 8] + s2[P // 8:]
            agg_scr[k:k + 1, :] = jnp.sum(s3.astype(jnp.float32),
                                          axis=0, keepdims=True)
        diag = jnp.maximum(jnp.transpose(a2bf) + cTbf, jnp.bfloat16(0))
        aggT = agg_scr[...] - diag.astype(jnp.float32)

        newT = (_dot(Wn1T_ref[...], aggT)
                + _dot(Wn2T_ref[...], nT)
                + bnc_ref[...])
        out_ref[i] = newT


def kernel(t, h, W_e, b_e, W_n, b_n):
    del t
    B = h.shape[0]
    nodes = h.reshape(B, P, D)
    nodesT = nodes.transpose(0, 2, 1)

    We1 = W_e[:D]
    We1T = We1.T
    We2T = W_e[D:2 * D].T
    wdbf = W_e[2 * D:2 * D + 1]
    be = b_e.reshape(1, ED)
    bec = b_e.reshape(ED, 1)
    Wn1T = W_n[:ED].T
    Wn2T = W_n[ED:].T
    bnc = b_n.reshape(D, 1)

    full = lambda shape: pl.BlockSpec(shape, lambda b: (0,) * len(shape))
    outT = pl.pallas_call(
        _gin_kernel,
        grid=(B // BB,),
        in_specs=[
            pl.BlockSpec((BB, P, D), lambda b: (b, 0, 0)),
            pl.BlockSpec((BB, D, P), lambda b: (b, 0, 0)),
            full((D, ED)), full((ED, D)), full((ED, D)), full((1, ED)),
            full((1, ED)), full((ED, 1)), full((D, ED)), full((D, D)),
            full((D, 1)),
        ],
        out_specs=pl.BlockSpec((BB, D, P), lambda b: (b, 0, 0)),
        out_shape=jax.ShapeDtypeStruct((B, D, P), jnp.float32),
        scratch_shapes=[pltpu.VMEM((ED, P), jnp.float32)],
        compiler_params=pltpu.CompilerParams(
            dimension_semantics=("parallel",)),
    )(nodes, nodesT, We1, We1T, We2T, wdbf, be, bec, Wn1T, Wn2T, bnc)

    return outT.transpose(0, 2, 1).reshape(B, P * D)


# speedup vs baseline: 3.3294x; 1.0288x over previous
import jax
import jax.numpy as jnp
from jax.experimental import pallas as pl
from jax.experimental.pallas import tpu as pltpu

P = 256
D = 16
ED = 16
BB = 2


def _dot(a, b, dims=((1,), (0,))):
    return jax.lax.dot_general(a, b, (dims, ((), ())),
                               preferred_element_type=jnp.float32)


def _gin_kernel(nodes_ref, nodesT_ref, We1_ref, We1T_ref, We2T_ref, wdbf_ref,
                be_ref, bec_ref, Wn1T_ref, Wn2T_ref, bnc_ref, out_ref, agg_scr):
    for i in range(BB):
        nodes = nodes_ref[i]
        nT = nodesT_ref[i]

        g = _dot(nodes, nT)
        sq_row = jnp.sum(nT * nT, axis=0, keepdims=True)
        sq_col = jnp.sum(nodes * nodes, axis=1, keepdims=True)
        d2 = jnp.abs(sq_col + sq_row - 2.0 * g)
        dist = d2 * jax.lax.rsqrt(d2 + 1e-30)
        distbf = dist.astype(jnp.bfloat16)

        a2bf = (_dot(nodes, We1_ref[...]) + be_ref[...]).astype(jnp.bfloat16)
        cTbf = _dot(We2T_ref[...], nT).astype(jnp.bfloat16)

        for k in range(ED):
            wk = jax.lax.convert_element_type(wdbf_ref[0, k], jnp.bfloat16)
            m = distbf * wk + a2bf[:, k:k + 1] + cTbf[k:k + 1, :]
            m = jnp.maximum(m, jnp.bfloat16(0))
            s1 = m[:P // 2] + m[P // 2:]
            s2 = s1[:P // 4] + s1[P // 4:]
            s3 = s2[:P // 8] + s2[P // 8:]
            s4 = s3[:P // 16] + s3[P // 16:]
            agg_scr[k:k + 1, :] = jnp.sum(s4.astype(jnp.float32),
                                          axis=0, keepdims=True)
        diag = jnp.maximum(jnp.transpose(a2bf) + cTbf, jnp.bfloat16(0))
        aggT = agg_scr[...] - diag.astype(jnp.float32)

        newT = (_dot(Wn1T_ref[...], aggT)
                + _dot(Wn2T_ref[...], nT)
                + bnc_ref[...])
        out_ref[i] = newT


def kernel(t, h, W_e, b_e, W_n, b_n):
    del t
    B = h.shape[0]
    nodes = h.reshape(B, P, D)
    nodesT = nodes.transpose(0, 2, 1)

    We1 = W_e[:D]
    We1T = We1.T
    We2T = W_e[D:2 * D].T
    wdbf = W_e[2 * D:2 * D + 1]
    be = b_e.reshape(1, ED)
    bec = b_e.reshape(ED, 1)
    Wn1T = W_n[:ED].T
    Wn2T = W_n[ED:].T
    bnc = b_n.reshape(D, 1)

    full = lambda shape: pl.BlockSpec(shape, lambda b: (0,) * len(shape))
    outT = pl.pallas_call(
        _gin_kernel,
        grid=(B // BB,),
        in_specs=[
            pl.BlockSpec((BB, P, D), lambda b: (b, 0, 0)),
            pl.BlockSpec((BB, D, P), lambda b: (b, 0, 0)),
            full((D, ED)), full((ED, D)), full((ED, D)), full((1, ED)),
            full((1, ED)), full((ED, 1)), full((D, ED)), full((D, D)),
            full((D, 1)),
        ],
        out_specs=pl.BlockSpec((BB, D, P), lambda b: (b, 0, 0)),
        out_shape=jax.ShapeDtypeStruct((B, D, P), jnp.float32),
        scratch_shapes=[pltpu.VMEM((ED, P), jnp.float32)],
        compiler_params=pltpu.CompilerParams(
            dimension_semantics=("parallel",)),
    )(nodes, nodesT, We1, We1T, We2T, wdbf, be, bec, Wn1T, Wn2T, bnc)

    return outT.transpose(0, 2, 1).reshape(B, P * D)


# speedup vs baseline: 3.3675x; 1.0114x over previous
import jax
import jax.numpy as jnp
from jax.experimental import pallas as pl
from jax.experimental.pallas import tpu as pltpu

P = 256
D = 16
ED = 16
BB = 2


def _dot(a, b, dims=((1,), (0,))):
    return jax.lax.dot_general(a, b, (dims, ((), ())),
                               preferred_element_type=jnp.float32)


def _gin_kernel(nodes_ref, nodesT_ref, We1_ref, We1T_ref, We2T_ref, wdbf_ref,
                be_ref, bec_ref, Wn1T_ref, Wn2T_ref, bnc_ref, out_ref):
    for i in range(BB):
        nodes = nodes_ref[i]
        nT = nodesT_ref[i]

        g = _dot(nodes, nT)
        sq_row = jnp.sum(nT * nT, axis=0, keepdims=True)
        sq_col = jnp.sum(nodes * nodes, axis=1, keepdims=True)
        d2 = jnp.abs(sq_col + sq_row - 2.0 * g)
        dist = d2 * jax.lax.rsqrt(d2 + 1e-30)
        distbf = dist.astype(jnp.bfloat16)

        a2bf = (_dot(nodes, We1_ref[...]) + be_ref[...]).astype(jnp.bfloat16)
        cTbf = _dot(We2T_ref[...], nT).astype(jnp.bfloat16)

        rows = []
        for k in range(ED):
            wk = jax.lax.convert_element_type(wdbf_ref[0, k], jnp.bfloat16)
            m = distbf * wk + a2bf[:, k:k + 1] + cTbf[k:k + 1, :]
            m = jnp.maximum(m, jnp.bfloat16(0))
            s1 = m[:P // 2] + m[P // 2:]
            s2 = s1[:P // 4] + s1[P // 4:]
            s3 = s2[:P // 8] + s2[P // 8:]
            s4 = s3[:P // 16] + s3[P // 16:]
            rows.append(jnp.sum(s4.astype(jnp.float32),
                                axis=0, keepdims=True))
        diag = jnp.maximum(jnp.transpose(a2bf) + cTbf, jnp.bfloat16(0))
        aggT = jnp.concatenate(rows, axis=0) - diag.astype(jnp.float32)

        newT = (_dot(Wn1T_ref[...], aggT)
                + _dot(Wn2T_ref[...], nT)
                + bnc_ref[...])
        out_ref[i] = newT


def kernel(t, h, W_e, b_e, W_n, b_n):
    del t
    B = h.shape[0]
    nodes = h.reshape(B, P, D)
    nodesT = nodes.transpose(0, 2, 1)

    We1 = W_e[:D]
    We1T = We1.T
    We2T = W_e[D:2 * D].T
    wdbf = W_e[2 * D:2 * D + 1]
    be = b_e.reshape(1, ED)
    bec = b_e.reshape(ED, 1)
    Wn1T = W_n[:ED].T
    Wn2T = W_n[ED:].T
    bnc = b_n.reshape(D, 1)

    full = lambda shape: pl.BlockSpec(shape, lambda b: (0,) * len(shape))
    outT = pl.pallas_call(
        _gin_kernel,
        grid=(B // BB,),
        in_specs=[
            pl.BlockSpec((BB, P, D), lambda b: (b, 0, 0)),
            pl.BlockSpec((BB, D, P), lambda b: (b, 0, 0)),
            full((D, ED)), full((ED, D)), full((ED, D)), full((1, ED)),
            full((1, ED)), full((ED, 1)), full((D, ED)), full((D, D)),
            full((D, 1)),
        ],
        out_specs=pl.BlockSpec((BB, D, P), lambda b: (b, 0, 0)),
        out_shape=jax.ShapeDtypeStruct((B, D, P), jnp.float32),
        compiler_params=pltpu.CompilerParams(
            dimension_semantics=("parallel",)),
    )(nodes, nodesT, We1, We1T, We2T, wdbf, be, bec, Wn1T, Wn2T, bnc)

    return outT.transpose(0, 2, 1).reshape(B, P * D)


# speedup vs baseline: 3.4681x; 1.0299x over previous
import jax
import jax.numpy as jnp
from jax.experimental import pallas as pl
from jax.experimental.pallas import tpu as pltpu

P = 256
D = 16
ED = 16
BB = 2


def _dot(a, b, dims=((1,), (0,))):
    return jax.lax.dot_general(a, b, (dims, ((), ())),
                               preferred_element_type=jnp.float32)


def _gin_kernel(nodes_ref, nodesT_ref, We1_ref, We1T_ref, We2T_ref, wdbf_ref,
                be_ref, bec_ref, Wn1T_ref, Wn2T_ref, bnc_ref, out_ref):
    pre = []
    for i in range(BB):
        nodes = nodes_ref[i]
        nT = nodesT_ref[i]

        g = _dot(nodes, nT)
        sq_row = jnp.sum(nT * nT, axis=0, keepdims=True)
        sq_col = jnp.sum(nodes * nodes, axis=1, keepdims=True)
        d2 = jnp.abs(sq_col + sq_row - 2.0 * g)
        dist = d2 * jax.lax.rsqrt(d2 + 1e-30)
        distbf = dist.astype(jnp.bfloat16)

        a2bf = (_dot(nodes, We1_ref[...]) + be_ref[...]).astype(jnp.bfloat16)
        cTbf = _dot(We2T_ref[...], nT).astype(jnp.bfloat16)
        pre.append((nT, distbf, a2bf, cTbf))

    for i in range(BB):
        nT, distbf, a2bf, cTbf = pre[i]
        rows = []
        for k in range(ED):
            wk = jax.lax.convert_element_type(wdbf_ref[0, k], jnp.bfloat16)
            m = distbf * wk + a2bf[:, k:k + 1] + cTbf[k:k + 1, :]
            m = jnp.maximum(m, jnp.bfloat16(0))
            s1 = m[:P // 2] + m[P // 2:]
            s2 = s1[:P // 4] + s1[P // 4:]
            s3 = s2[:P // 8] + s2[P // 8:]
            s4 = s3[:P // 16] + s3[P // 16:]
            rows.append(jnp.sum(s4.astype(jnp.float32),
                                axis=0, keepdims=True))
        diag = jnp.maximum(jnp.transpose(a2bf) + cTbf, jnp.bfloat16(0))
        aggT = jnp.concatenate(rows, axis=0) - diag.astype(jnp.float32)

        newT = (_dot(Wn1T_ref[...], aggT)
                + _dot(Wn2T_ref[...], nT)
                + bnc_ref[...])
        out_ref[i] = newT


def kernel(t, h, W_e, b_e, W_n, b_n):
    del t
    B = h.shape[0]
    nodes = h.reshape(B, P, D)
    nodesT = nodes.transpose(0, 2, 1)

    We1 = W_e[:D]
    We1T = We1.T
    We2T = W_e[D:2 * D].T
    wdbf = W_e[2 * D:2 * D + 1]
    be = b_e.reshape(1, ED)
    bec = b_e.reshape(ED, 1)
    Wn1T = W_n[:ED].T
    Wn2T = W_n[ED:].T
    bnc = b_n.reshape(D, 1)

    full = lambda shape: pl.BlockSpec(shape, lambda b: (0,) * len(shape))
    outT = pl.pallas_call(
        _gin_kernel,
        grid=(B // BB,),
        in_specs=[
            pl.BlockSpec((BB, P, D), lambda b: (b, 0, 0)),
            pl.BlockSpec((BB, D, P), lambda b: (b, 0, 0)),
            full((D, ED)), full((ED, D)), full((ED, D)), full((1, ED)),
            full((1, ED)), full((ED, 1)), full((D, ED)), full((D, D)),
            full((D, 1)),
        ],
        out_specs=pl.BlockSpec((BB, D, P), lambda b: (b, 0, 0)),
        out_shape=jax.ShapeDtypeStruct((B, D, P), jnp.float32),
        compiler_params=pltpu.CompilerParams(
            dimension_semantics=("parallel",)),
    )(nodes, nodesT, We1, We1T, We2T, wdbf, be, bec, Wn1T, Wn2T, bnc)

    return outT.transpose(0, 2, 1).reshape(B, P * D)
